# baseline (speedup 1.0000x reference)
import functools

import jax
import jax.numpy as jnp
from jax import lax
from jax.experimental import pallas as pl
from jax.experimental.pallas import tpu as pltpu
from jax.experimental.pallas import tpu_sc as plsc

_EPS = 1e-12
_NC = 2
_NS = 16
_NW = _NC * _NS
_L = 16
_CH = 16


def _allreduce_sum16(v, scratch):
    lanes = lax.iota(jnp.int32, _L)
    for sh in (8, 4, 2, 1):
        scratch[...] = v
        v = v + plsc.load_gather(scratch, [lanes ^ sh])
    return v


def _rsqrt16(v):
    ii = plsc.bitcast(v, jnp.int32)
    magic = jnp.full((_L,), 0x5F3759DF, jnp.int32)
    r = plsc.bitcast(magic - lax.shift_right_logical(ii, 1), jnp.float32)
    for _ in range(4):
        r = r * (1.5 - 0.5 * v * r * r)
    return r


def _make_sc_call(R, S, D):
    rw = R // _NW
    nch = rw // _CH
    nj = D // _L
    mesh = plsc.VectorSubcoreMesh(core_axis_name="c", subcore_axis_name="s")

    @functools.partial(
        pl.kernel,
        mesh=mesh,
        out_type=jax.ShapeDtypeStruct((R * D,), jnp.float32),
        compiler_params=pltpu.CompilerParams(needs_layout_passes=False),
        scratch_types=[
            pltpu.VMEM((_CH * D,), jnp.float32),
            pltpu.VMEM((_CH * D,), jnp.float32),
            pltpu.VMEM((_CH * D,), jnp.float32),
            pltpu.VMEM((D,), jnp.float32),
            pltpu.VMEM((D,), jnp.float32),
            pltpu.VMEM((_L,), jnp.float32),
        ],
    )
    def sc_call(x_hbm, t_hbm, g_hbm, b_hbm, o_hbm, xb, tb, ob, gb, bb, rs):
        cid = lax.axis_index("c")
        sid = lax.axis_index("s")
        wid = sid * _NC + cid
        base = wid * rw
        pltpu.sync_copy(g_hbm, gb)
        pltpu.sync_copy(b_hbm, bb)

        def row_body(i, _):
            roff = i * D

            def p1(j, carry):
                acc, acc2 = carry
                sl = pl.ds(roff + j * _L, _L)
                e = xb[sl] + tb[sl]
                xb[sl] = e
                return acc + e, acc2 + e * e

            zero = jnp.zeros((_L,), jnp.float32)
            acc, acc2 = lax.fori_loop(0, nj, p1, (zero, zero), unroll=8)
            meanv = _allreduce_sum16(acc, rs) * (1.0 / D)
            varv = _allreduce_sum16(acc2, rs) * (1.0 / D) - meanv * meanv
            varv = jnp.maximum(varv, 0.0)
            stdv = varv * _rsqrt16(varv)
            invv = _rsqrt16(stdv + _EPS)

            def p2(j, _):
                sl = pl.ds(roff + j * _L, _L)
                gl = pl.ds(j * _L, _L)
                ob[sl] = gb[gl] * ((xb[sl] - meanv) * invv) + bb[gl]
                return 0

            lax.fori_loop(0, nj, p2, 0, unroll=8)
            return 0

        def chunk_body(k, _):
            row0 = base + k * _CH
            s0 = lax.rem(row0, S)
            pltpu.sync_copy(x_hbm.at[pl.ds(row0 * D, _CH * D)], xb)
            pltpu.sync_copy(t_hbm.at[pl.ds(s0 * D, _CH * D)], tb)
            lax.fori_loop(0, _CH, row_body, 0)
            pltpu.sync_copy(ob, o_hbm.at[pl.ds(row0 * D, _CH * D)])
            return 0

        lax.fori_loop(0, nch, chunk_body, 0)

    return sc_call


def kernel(input, pos_table, gamma, beta):
    B, S, D = input.shape
    x = input.reshape(B * S * D)
    t = pos_table.reshape(S * D)
    out = _make_sc_call(B * S, S, D)(x, t, gamma, beta)
    return out.reshape(B, S, D)


# speedup vs baseline: 1.2758x; 1.2758x over previous
import functools

import jax
import jax.numpy as jnp
from jax import lax
from jax.experimental import pallas as pl
from jax.experimental.pallas import tpu as pltpu
from jax.experimental.pallas import tpu_sc as plsc

_EPS = 1e-12
_NC = 2
_NS = 16
_NW = _NC * _NS
_L = 16
_CH = 16


def _allreduce_sum16(v, scratch):
    lanes = lax.iota(jnp.int32, _L)
    for sh in (8, 4, 2, 1):
        scratch[...] = v
        v = v + plsc.load_gather(scratch, [lanes ^ sh])
    return v


def _rsqrt16(v):
    ii = plsc.bitcast(v, jnp.int32)
    magic = jnp.full((_L,), 0x5F3759DF, jnp.int32)
    r = plsc.bitcast(magic - lax.shift_right_logical(ii, 1), jnp.float32)
    for _ in range(4):
        r = r * (1.5 - 0.5 * v * r * r)
    return r


def _make_sc_call(R, S, D):
    rw = R // _NW
    nch = rw // _CH
    ng = nch // 2
    nj = D // _L
    cd = _CH * D
    mesh = plsc.VectorSubcoreMesh(core_axis_name="c", subcore_axis_name="s")

    @functools.partial(
        pl.kernel,
        mesh=mesh,
        out_type=jax.ShapeDtypeStruct((R * D,), jnp.float32),
        compiler_params=pltpu.CompilerParams(needs_layout_passes=False),
        scratch_types=[
            pltpu.VMEM((2 * cd,), jnp.float32),
            pltpu.VMEM((2 * cd,), jnp.float32),
            pltpu.VMEM((2 * cd,), jnp.float32),
            pltpu.VMEM((D,), jnp.float32),
            pltpu.VMEM((D,), jnp.float32),
            pltpu.VMEM((_L,), jnp.float32),
            pltpu.SemaphoreType.DMA,
            pltpu.SemaphoreType.DMA,
            pltpu.SemaphoreType.DMA,
            pltpu.SemaphoreType.DMA,
            pltpu.SemaphoreType.DMA,
            pltpu.SemaphoreType.DMA,
        ],
    )
    def sc_call(x_hbm, t_hbm, g_hbm, b_hbm, o_hbm, xb, tb, ob, gb, bb, rs,
                xs0, ts0, os0, xs1, ts1, os1):
        cid = lax.axis_index("c")
        sid = lax.axis_index("s")
        wid = sid * _NC + cid
        base = wid * rw
        pltpu.sync_copy(g_hbm, gb)
        pltpu.sync_copy(b_hbm, bb)

        def in_slices(chunk):
            row0 = base + chunk * _CH
            s0 = lax.rem(row0, S)
            return (x_hbm.at[pl.ds(row0 * D, cd)], t_hbm.at[pl.ds(s0 * D, cd)],
                    o_hbm.at[pl.ds(row0 * D, cd)])

        def issue_in(chunk, soff, xs, ts):
            xsl, tsl, _ = in_slices(chunk)
            pltpu.async_copy(xsl, xb.at[pl.ds(soff, cd)], xs)
            pltpu.async_copy(tsl, tb.at[pl.ds(soff, cd)], ts)

        def compute_chunk(soff):
            def row_body(i, _):
                roff = soff + i * D

                def p1(j, carry):
                    acc, acc2 = carry
                    sl = pl.ds(roff + j * _L, _L)
                    e = xb[sl] + tb[sl]
                    xb[sl] = e
                    return acc + e, acc2 + e * e

                zero = jnp.zeros((_L,), jnp.float32)
                acc, acc2 = lax.fori_loop(0, nj, p1, (zero, zero), unroll=8)
                meanv = _allreduce_sum16(acc, rs) * (1.0 / D)
                varv = _allreduce_sum16(acc2, rs) * (1.0 / D) - meanv * meanv
                varv = jnp.maximum(varv, 0.0)
                stdv = varv * _rsqrt16(varv)
                invv = _rsqrt16(stdv + _EPS)

                def p2(j, _):
                    sl = pl.ds(roff + j * _L, _L)
                    gl = pl.ds(j * _L, _L)
                    ob[sl] = gb[gl] * ((xb[sl] - meanv) * invv) + bb[gl]
                    return 0

                lax.fori_loop(0, nj, p2, 0, unroll=8)
                return 0

            lax.fori_loop(0, _CH, row_body, 0)

        def process(chunk, g, soff, xs, ts, os):
            xsl, tsl, osl = in_slices(chunk)
            pltpu.make_async_copy(xsl, xb.at[pl.ds(soff, cd)], xs).wait()
            pltpu.make_async_copy(tsl, tb.at[pl.ds(soff, cd)], ts).wait()

            @pl.when(g > 0)
            def _():
                pltpu.make_async_copy(ob.at[pl.ds(soff, cd)], osl, os).wait()

            compute_chunk(soff)
            pltpu.async_copy(ob.at[pl.ds(soff, cd)], osl, os)

        issue_in(0, 0, xs0, ts0)

        def ring(g, _):
            c0 = 2 * g
            c1 = c0 + 1
            issue_in(c1, cd, xs1, ts1)
            process(c0, g, 0, xs0, ts0, os0)

            @pl.when(g < ng - 1)
            def _():
                issue_in(c0 + 2, 0, xs0, ts0)

            process(c1, g, cd, xs1, ts1, os1)
            return 0

        lax.fori_loop(0, ng, ring, 0)
        _, _, osl0 = in_slices(nch - 2)
        _, _, osl1 = in_slices(nch - 1)
        pltpu.make_async_copy(ob.at[pl.ds(0, cd)], osl0, os0).wait()
        pltpu.make_async_copy(ob.at[pl.ds(cd, cd)], osl1, os1).wait()

    return sc_call


def kernel(input, pos_table, gamma, beta):
    B, S, D = input.shape
    x = input.reshape(B * S * D)
    t = pos_table.reshape(S * D)
    out = _make_sc_call(B * S, S, D)(x, t, gamma, beta)
    return out.reshape(B, S, D)


# speedup vs baseline: 1.2803x; 1.0035x over previous
import functools

import jax
import jax.numpy as jnp
from jax import lax
from jax.experimental import pallas as pl
from jax.experimental.pallas import tpu as pltpu
from jax.experimental.pallas import tpu_sc as plsc

_EPS = 1e-12
_NC = 2
_NS = 16
_NW = _NC * _NS
_L = 16
_CH = 16


def _allreduce_sum16(v, scratch):
    lanes = lax.iota(jnp.int32, _L)
    for sh in (8, 4, 2, 1):
        scratch[...] = v
        v = v + plsc.load_gather(scratch, [lanes ^ sh])
    return v


def _rsqrt16(v):
    ii = plsc.bitcast(v, jnp.int32)
    magic = jnp.full((_L,), 0x5F3759DF, jnp.int32)
    r = plsc.bitcast(magic - lax.shift_right_logical(ii, 1), jnp.float32)
    for _ in range(4):
        r = r * (1.5 - 0.5 * v * r * r)
    return r


def _make_sc_call(R, S, D):
    rw = R // _NW
    nch = rw // _CH
    ng = nch // 2
    nj = D // _L
    cd = _CH * D
    mesh = plsc.VectorSubcoreMesh(core_axis_name="c", subcore_axis_name="s")

    @functools.partial(
        pl.kernel,
        mesh=mesh,
        out_type=jax.ShapeDtypeStruct((R * D,), jnp.float32),
        compiler_params=pltpu.CompilerParams(needs_layout_passes=False),
        scratch_types=[
            pltpu.VMEM((2 * cd,), jnp.float32),
            pltpu.VMEM((2 * cd,), jnp.float32),
            pltpu.VMEM((2 * cd,), jnp.float32),
            pltpu.VMEM((D,), jnp.float32),
            pltpu.VMEM((D,), jnp.float32),
            pltpu.VMEM((_L,), jnp.float32),
            pltpu.SemaphoreType.DMA,
            pltpu.SemaphoreType.DMA,
            pltpu.SemaphoreType.DMA,
            pltpu.SemaphoreType.DMA,
            pltpu.SemaphoreType.DMA,
            pltpu.SemaphoreType.DMA,
        ],
    )
    def sc_call(x_hbm, t_hbm, g_hbm, b_hbm, o_hbm, xb, tb, ob, gb, bb, rs,
                xs0, ts0, os0, xs1, ts1, os1):
        cid = lax.axis_index("c")
        sid = lax.axis_index("s")
        wid = sid * _NC + cid
        base = wid * rw
        pltpu.sync_copy(g_hbm, gb)
        pltpu.sync_copy(b_hbm, bb)

        def in_slices(chunk):
            row0 = base + chunk * _CH
            s0 = lax.rem(row0, S)
            return (x_hbm.at[pl.ds(row0 * D, cd)], t_hbm.at[pl.ds(s0 * D, cd)],
                    o_hbm.at[pl.ds(row0 * D, cd)])

        def issue_in(chunk, soff, xs, ts):
            xsl, tsl, _ = in_slices(chunk)
            pltpu.async_copy(xsl, xb.at[pl.ds(soff, cd)], xs)
            pltpu.async_copy(tsl, tb.at[pl.ds(soff, cd)], ts)

        def compute_chunk(soff):
            def row_body(i, _):
                roff = soff + i * D

                def p1(j, carry):
                    accs = list(carry)
                    j0 = roff + j * (4 * _L)
                    for u in range(4):
                        sl = pl.ds(j0 + u * _L, _L)
                        e = xb[sl] + tb[sl]
                        xb[sl] = e
                        accs[u] = accs[u] + e
                        accs[4 + u] = accs[4 + u] + e * e
                    return tuple(accs)

                zero = jnp.zeros((_L,), jnp.float32)
                accs = lax.fori_loop(0, nj // 4, p1, (zero,) * 8, unroll=4)
                acc = (accs[0] + accs[1]) + (accs[2] + accs[3])
                acc2 = (accs[4] + accs[5]) + (accs[6] + accs[7])
                meanv = _allreduce_sum16(acc, rs) * (1.0 / D)
                varv = _allreduce_sum16(acc2, rs) * (1.0 / D) - meanv * meanv
                varv = jnp.maximum(varv, 0.0)
                stdv = varv * _rsqrt16(varv)
                invv = _rsqrt16(stdv + _EPS)

                def p2(j, _):
                    sl = pl.ds(roff + j * _L, _L)
                    gl = pl.ds(j * _L, _L)
                    ob[sl] = gb[gl] * ((xb[sl] - meanv) * invv) + bb[gl]
                    return 0

                lax.fori_loop(0, nj, p2, 0, unroll=8)
                return 0

            lax.fori_loop(0, _CH, row_body, 0)

        def process(chunk, g, soff, xs, ts, os):
            xsl, tsl, osl = in_slices(chunk)
            pltpu.make_async_copy(xsl, xb.at[pl.ds(soff, cd)], xs).wait()
            pltpu.make_async_copy(tsl, tb.at[pl.ds(soff, cd)], ts).wait()

            @pl.when(g > 0)
            def _():
                pltpu.make_async_copy(ob.at[pl.ds(soff, cd)], osl, os).wait()

            compute_chunk(soff)
            pltpu.async_copy(ob.at[pl.ds(soff, cd)], osl, os)

        issue_in(0, 0, xs0, ts0)

        def ring(g, _):
            c0 = 2 * g
            c1 = c0 + 1
            issue_in(c1, cd, xs1, ts1)
            process(c0, g, 0, xs0, ts0, os0)

            @pl.when(g < ng - 1)
            def _():
                issue_in(c0 + 2, 0, xs0, ts0)

            process(c1, g, cd, xs1, ts1, os1)
            return 0

        lax.fori_loop(0, ng, ring, 0)
        _, _, osl0 = in_slices(nch - 2)
        _, _, osl1 = in_slices(nch - 1)
        pltpu.make_async_copy(ob.at[pl.ds(0, cd)], osl0, os0).wait()
        pltpu.make_async_copy(ob.at[pl.ds(cd, cd)], osl1, os1).wait()

    return sc_call


def kernel(input, pos_table, gamma, beta):
    B, S, D = input.shape
    x = input.reshape(B * S * D)
    t = pos_table.reshape(S * D)
    out = _make_sc_call(B * S, S, D)(x, t, gamma, beta)
    return out.reshape(B, S, D)


# speedup vs baseline: 2.5095x; 1.9601x over previous
import functools

import jax
import jax.numpy as jnp
from jax import lax
from jax.experimental import pallas as pl
from jax.experimental.pallas import tpu as pltpu
from jax.experimental.pallas import tpu_sc as plsc

_EPS = 1e-12
_NC = 2
_NS = 16
_NW = _NC * _NS
_L = 16
_CH = 16


def _allreduce_sum16(v, scratch):
    lanes = lax.iota(jnp.int32, _L)
    for sh in (8, 4, 2, 1):
        scratch[...] = v
        v = v + plsc.load_gather(scratch, [lanes ^ sh])
    return v


def _rsqrt16(v):
    ii = plsc.bitcast(v, jnp.int32)
    magic = jnp.full((_L,), 0x5F3759DF, jnp.int32)
    r = plsc.bitcast(magic - lax.shift_right_logical(ii, 1), jnp.float32)
    for _ in range(4):
        r = r * (1.5 - 0.5 * v * r * r)
    return r


def _make_sc_call(R, S, D):
    rw = R // _NW
    nch = rw // _CH
    ng = nch // 2
    nj = D // _L
    cd = _CH * D
    mesh = plsc.VectorSubcoreMesh(core_axis_name="c", subcore_axis_name="s")

    @functools.partial(
        pl.kernel,
        mesh=mesh,
        out_type=jax.ShapeDtypeStruct((R * D,), jnp.float32),
        compiler_params=pltpu.CompilerParams(needs_layout_passes=False),
        scratch_types=[
            pltpu.VMEM((2 * cd,), jnp.float32),
            pltpu.VMEM((2 * cd,), jnp.float32),
            pltpu.VMEM((2 * cd,), jnp.float32),
            pltpu.VMEM((D,), jnp.float32),
            pltpu.VMEM((D,), jnp.float32),
            pltpu.VMEM((_L,), jnp.float32),
            pltpu.SemaphoreType.DMA,
            pltpu.SemaphoreType.DMA,
            pltpu.SemaphoreType.DMA,
            pltpu.SemaphoreType.DMA,
            pltpu.SemaphoreType.DMA,
            pltpu.SemaphoreType.DMA,
        ],
    )
    def sc_call(x_hbm, t_hbm, g_hbm, b_hbm, o_hbm, xb, tb, ob, gb, bb, rs,
                xs0, ts0, os0, xs1, ts1, os1):
        cid = lax.axis_index("c")
        sid = lax.axis_index("s")
        wid = sid * _NC + cid
        base = wid * rw
        pltpu.sync_copy(g_hbm, gb)
        pltpu.sync_copy(b_hbm, bb)

        def in_slices(chunk):
            row0 = base + chunk * _CH
            s0 = lax.rem(row0, S)
            return (x_hbm.at[pl.ds(row0 * D, cd)], t_hbm.at[pl.ds(s0 * D, cd)],
                    o_hbm.at[pl.ds(row0 * D, cd)])

        def issue_in(chunk, soff, xs, ts):
            xsl, tsl, _ = in_slices(chunk)
            pltpu.async_copy(xsl, xb.at[pl.ds(soff, cd)], xs)
            pltpu.async_copy(tsl, tb.at[pl.ds(soff, cd)], ts)

        def compute_chunk(soff):
            def row_body(i, _):
                roff = soff + i * D

                zero = jnp.zeros((_L,), jnp.float32)

                @plsc.parallel_loop(0, nj // 4, carry=(zero,) * 8, unroll=2)
                def accs(j, carry):
                    a = list(carry)
                    j0 = roff + j * (4 * _L)
                    for u in range(4):
                        sl = pl.ds(j0 + u * _L, _L)
                        e = xb[sl] + tb[sl]
                        xb[sl] = e
                        a[u] = a[u] + e
                        a[4 + u] = a[4 + u] + e * e
                    return tuple(a)

                acc = (accs[0] + accs[1]) + (accs[2] + accs[3])
                acc2 = (accs[4] + accs[5]) + (accs[6] + accs[7])
                meanv = _allreduce_sum16(acc, rs) * (1.0 / D)
                varv = _allreduce_sum16(acc2, rs) * (1.0 / D) - meanv * meanv
                varv = jnp.maximum(varv, 0.0)
                stdv = varv * _rsqrt16(varv)
                invv = _rsqrt16(stdv + _EPS)

                @plsc.parallel_loop(0, nj, unroll=4)
                def _(j):
                    sl = pl.ds(roff + j * _L, _L)
                    gl = pl.ds(j * _L, _L)
                    ob[sl] = gb[gl] * ((xb[sl] - meanv) * invv) + bb[gl]

                return 0

            lax.fori_loop(0, _CH, row_body, 0)

        def process(chunk, g, soff, xs, ts, os):
            xsl, tsl, osl = in_slices(chunk)
            pltpu.make_async_copy(xsl, xb.at[pl.ds(soff, cd)], xs).wait()
            pltpu.make_async_copy(tsl, tb.at[pl.ds(soff, cd)], ts).wait()

            @pl.when(g > 0)
            def _():
                pltpu.make_async_copy(ob.at[pl.ds(soff, cd)], osl, os).wait()

            compute_chunk(soff)
            pltpu.async_copy(ob.at[pl.ds(soff, cd)], osl, os)

        issue_in(0, 0, xs0, ts0)

        def ring(g, _):
            c0 = 2 * g
            c1 = c0 + 1
            issue_in(c1, cd, xs1, ts1)
            process(c0, g, 0, xs0, ts0, os0)

            @pl.when(g < ng - 1)
            def _():
                issue_in(c0 + 2, 0, xs0, ts0)

            process(c1, g, cd, xs1, ts1, os1)
            return 0

        lax.fori_loop(0, ng, ring, 0)
        _, _, osl0 = in_slices(nch - 2)
        _, _, osl1 = in_slices(nch - 1)
        pltpu.make_async_copy(ob.at[pl.ds(0, cd)], osl0, os0).wait()
        pltpu.make_async_copy(ob.at[pl.ds(cd, cd)], osl1, os1).wait()

    return sc_call


def kernel(input, pos_table, gamma, beta):
    B, S, D = input.shape
    x = input.reshape(B * S * D)
    t = pos_table.reshape(S * D)
    out = _make_sc_call(B * S, S, D)(x, t, gamma, beta)
    return out.reshape(B, S, D)


# speedup vs baseline: 14.6469x; 5.8367x over previous
import jax
import jax.numpy as jnp
from jax.experimental import pallas as pl

_EPS = 1e-12


def _body(x_ref, p_ref, g_ref, b_ref, o_ref):
    x = x_ref[...]
    p = p_ref[...]
    emb = x + p[None]
    mean = jnp.mean(emb, axis=2, keepdims=True)
    d = emb - mean
    var = jnp.mean(d * d, axis=2, keepdims=True)
    std = jnp.sqrt(var)
    inv = jax.lax.rsqrt(std + _EPS)
    o_ref[...] = g_ref[...] * (d * inv) + b_ref[...]


def kernel(input, pos_table, gamma, beta):
    B, S, D = input.shape
    BS = 512
    BB = B
    grid = (S // BS, B // BB)
    out = pl.pallas_call(
        _body,
        grid=grid,
        in_specs=[
            pl.BlockSpec((BB, BS, D), lambda s, b: (b, s, 0)),
            pl.BlockSpec((BS, D), lambda s, b: (s, 0)),
            pl.BlockSpec((1, D), lambda s, b: (0, 0)),
            pl.BlockSpec((1, D), lambda s, b: (0, 0)),
        ],
        out_specs=pl.BlockSpec((BB, BS, D), lambda s, b: (b, s, 0)),
        out_shape=jax.ShapeDtypeStruct((B, S, D), jnp.float32),
    )(input, pos_table, gamma.reshape(1, D), beta.reshape(1, D))
    return out
